# baseline (speedup 1.0000x reference)
import functools

import jax
import jax.numpy as jnp
from jax import lax
from jax.experimental import pallas as pl
from jax.experimental.pallas import tpu as pltpu
from jax.experimental.pallas import tpu_sc as plsc

NC = 2
NS = 16
L = 16
NW = NC * NS
B = 16384
NEG = 25
PAIRS = NEG + 1
H = 64
HV = H // L
PW = B // NW * PAIRS
CH_ROWS = 4
CP = CH_ROWS * PAIRS
NCH = PW // CP
MARGIN = 1.0


def _sc_loss_call():
    mesh = plsc.VectorSubcoreMesh(
        core_axis_name="c", subcore_axis_name="s", num_cores=NC)

    @functools.partial(
        pl.kernel,
        mesh=mesh,
        compiler_params=pltpu.CompilerParams(
            use_tc_tiling_on_sc=False, needs_layout_passes=False),
        out_type=jax.ShapeDtypeStruct((NW * L,), jnp.float32),
        scratch_types=[
            pltpu.VMEM((PW,), jnp.int32),
            pltpu.VMEM((PW,), jnp.int32),
            pltpu.VMEM((PW,), jnp.int32),
            pltpu.VMEM((2, CP, H), jnp.float32),
            pltpu.VMEM((2, CP, H), jnp.float32),
            pltpu.VMEM((2, CP, H), jnp.float32),
            pltpu.VMEM((2, CP, H), jnp.float32),
            pltpu.VMEM((2, CP, H), jnp.float32),
            pltpu.VMEM((2, CP, H), jnp.float32),
            pltpu.VMEM((L,), jnp.float32),
            pltpu.SemaphoreType.DMA,
            pltpu.SemaphoreType.DMA,
        ],
    )
    def sc_loss(h_hbm, t_hbm, r_hbm, ent_e, rel_e, ent_t, rel_t, out_hbm,
                hidx, tidx, ridx, he, ht, te, tt, rt, re, outv, sem0, sem1):
        wid = lax.axis_index("s") * NC + lax.axis_index("c")

        pltpu.sync_copy(h_hbm.at[pl.ds(wid * PW, PW)], hidx)
        pltpu.sync_copy(t_hbm.at[pl.ds(wid * PW, PW)], tidx)
        pltpu.sync_copy(r_hbm.at[pl.ds(wid * PW, PW)], ridx)

        sems = (sem0, sem1)
        bufs = (he, ht, te, tt, rt, re)
        tabs = (ent_e, ent_t, ent_e, ent_t, rel_t, rel_e)
        idxs = (hidx, hidx, tidx, tidx, ridx, ridx)

        def start(g, slot):
            for buf, tab, ix in zip(bufs, tabs, idxs):
                pltpu.async_copy(
                    tab.at[ix.at[pl.ds(g * CP, CP)]], buf.at[slot],
                    sems[slot])

        def drain(slot):
            for buf in bufs:
                pltpu.make_async_copy(
                    ent_e.at[pl.ds(0, CP)], buf.at[slot],
                    sems[slot]).wait()

        def bsum(v):
            for step in (8, 4, 2, 1):
                idx = lax.iota(jnp.int32, L) ^ step
                v = v + v.at[idx].get(mode="promise_in_bounds")
            return v

        def pair_sv(slot, p):
            hev, tev, q = [], [], None
            for c in range(HV):
                hec = he[slot, p, pl.ds(c * L, L)]
                htc = ht[slot, p, pl.ds(c * L, L)]
                tec = te[slot, p, pl.ds(c * L, L)]
                ttc = tt[slot, p, pl.ds(c * L, L)]
                hev.append(hec)
                tev.append(tec)
                qc = hec * htc - tec * ttc
                q = qc if q is None else q + qc
            d = bsum(q)
            s = None
            for c in range(HV):
                rtc = rt[slot, p, pl.ds(c * L, L)]
                rec = re[slot, p, pl.ds(c * L, L)]
                sc = jnp.abs(hev[c] - tev[c] + d * rtc + rec)
                s = sc if s is None else s + sc
            return s

        def compute(g, slot, partial):
            del g

            def row_body(rr, acc):
                base = rr * PAIRS
                ps = pair_sv(slot, base)

                def neg_body(k, nacc):
                    return nacc + pair_sv(slot, base + k)

                nacc = lax.fori_loop(
                    1, PAIRS, neg_body, jnp.zeros((L,), jnp.float32))
                z = bsum(ps) - bsum(nacc) * (1.0 / NEG) + MARGIN
                return acc + jnp.maximum(z, 0.0)

            return lax.fori_loop(0, CH_ROWS, row_body, partial)

        start(0, 0)

        def outer(i, partial):
            g = 2 * i
            start(g + 1, 1)
            drain(0)
            partial = compute(g, 0, partial)

            @pl.when(g + 2 < NCH)
            def _():
                start(g + 2, 0)

            drain(1)
            return compute(g + 1, 1, partial)

        partial = lax.fori_loop(0, NCH // 2, outer,
                                jnp.zeros((L,), jnp.float32))
        outv[...] = partial
        pltpu.sync_copy(outv, out_hbm.at[pl.ds(wid * L, L)])

    return sc_loss


_SC_LOSS = _sc_loss_call()


def kernel(pos_h, pos_t, pos_r, neg_h, neg_t, neg_r,
           ent_embeddings, rel_embeddings, ent_transfer, rel_transfer):
    h_idx = jnp.concatenate(
        [pos_h.astype(jnp.int32), neg_h.astype(jnp.int32)], axis=1).reshape(-1)
    t_idx = jnp.concatenate(
        [pos_t.astype(jnp.int32), neg_t.astype(jnp.int32)], axis=1).reshape(-1)
    r_idx = jnp.concatenate(
        [pos_r.astype(jnp.int32), neg_r.astype(jnp.int32)], axis=1).reshape(-1)
    partials = _SC_LOSS(h_idx, t_idx, r_idx, ent_embeddings, rel_embeddings,
                        ent_transfer, rel_transfer)
    return jnp.sum(partials[::L])


# speedup vs baseline: 1.1143x; 1.1143x over previous
import functools

import jax
import jax.numpy as jnp
from jax import lax
from jax.experimental import pallas as pl
from jax.experimental.pallas import tpu as pltpu
from jax.experimental.pallas import tpu_sc as plsc

NC = 2
NS = 16
L = 16
NW = NC * NS
B = 16384
NEG = 25
PAIRS = NEG + 1
H = 64
HV = H // L
PW = B // NW * PAIRS
CH_ROWS = 4
CP = CH_ROWS * PAIRS
NCH = PW // CP
MARGIN = 1.0
ENT_TOTAL = 100000


def _sc_loss_call():
    mesh = plsc.VectorSubcoreMesh(
        core_axis_name="c", subcore_axis_name="s", num_cores=NC)

    @functools.partial(
        pl.kernel,
        mesh=mesh,
        compiler_params=pltpu.CompilerParams(
            use_tc_tiling_on_sc=False, needs_layout_passes=False),
        out_type=jax.ShapeDtypeStruct((NW * L,), jnp.float32),
        scratch_types=[
            pltpu.VMEM((PW,), jnp.int32),
            pltpu.VMEM((PW,), jnp.int32),
            pltpu.VMEM((PW,), jnp.int32),
            pltpu.VMEM((2, CP, H), jnp.int32),
            pltpu.VMEM((2, CP, H), jnp.int32),
            pltpu.VMEM((2, CP, H), jnp.int32),
            pltpu.VMEM((L,), jnp.float32),
            pltpu.SemaphoreType.DMA,
            pltpu.SemaphoreType.DMA,
        ],
    )
    def sc_loss(h_hbm, t_hbm, r_hbm, ent_pk, rel_pk, out_hbm,
                hidx, tidx, ridx, hb, tb, rb, outv, sem0, sem1):
        wid = lax.axis_index("s") * NC + lax.axis_index("c")

        pltpu.sync_copy(h_hbm.at[pl.ds(wid * PW, PW)], hidx)
        pltpu.sync_copy(t_hbm.at[pl.ds(wid * PW, PW)], tidx)
        pltpu.sync_copy(r_hbm.at[pl.ds(wid * PW, PW)], ridx)

        sems = (sem0, sem1)
        bufs = (hb, tb, rb)
        tabs = (ent_pk, ent_pk, rel_pk)
        idxs = (hidx, tidx, ridx)

        def start(g, slot):
            for buf, tab, ix in zip(bufs, tabs, idxs):
                pltpu.async_copy(
                    tab.at[ix.at[pl.ds(g * CP, CP)]], buf.at[slot],
                    sems[slot])

        def drain(slot):
            for buf in bufs:
                pltpu.make_async_copy(
                    ent_pk.at[pl.ds(0, CP)], buf.at[slot],
                    sems[slot]).wait()

        def bsum(v):
            for step in (8, 4, 2, 1):
                idx = lax.iota(jnp.int32, L) ^ step
                v = v + v.at[idx].get(mode="promise_in_bounds")
            return v

        def unpk(v):
            lo = plsc.bitcast(v << 16, jnp.float32)
            hi = plsc.bitcast(v & jnp.int32(-65536), jnp.float32)
            return lo, hi

        def pair_sv(slot, p):
            he, te, q = [], [], None
            for c in range(HV):
                hec, htc = unpk(hb[slot, p, pl.ds(c * L, L)])
                tec, ttc = unpk(tb[slot, p, pl.ds(c * L, L)])
                he.append(hec)
                te.append(tec)
                qc = hec * htc - tec * ttc
                q = qc if q is None else q + qc
            d = jnp.sum(q)
            s = None
            for c in range(HV):
                rtc, rec = unpk(rb[slot, p, pl.ds(c * L, L)])
                sc = jnp.abs(he[c] - te[c] + d * rtc + rec)
                s = sc if s is None else s + sc
            return s

        def compute(g, slot, partial):
            del g

            def row_body(rr, acc):
                base = rr * PAIRS
                ps = pair_sv(slot, base)

                def neg_body(k, nacc):
                    return nacc + pair_sv(slot, base + k)

                nacc = lax.fori_loop(
                    1, PAIRS, neg_body, jnp.zeros((L,), jnp.float32))
                z = bsum(ps) - bsum(nacc) * (1.0 / NEG) + MARGIN
                return acc + jnp.maximum(z, 0.0)

            return lax.fori_loop(0, CH_ROWS, row_body, partial)

        start(0, 0)

        def outer(i, partial):
            g = 2 * i
            start(g + 1, 1)
            drain(0)
            partial = compute(g, 0, partial)

            @pl.when(g + 2 < NCH)
            def _():
                start(g + 2, 0)

            drain(1)
            return compute(g + 1, 1, partial)

        partial = lax.fori_loop(0, NCH // 2, outer,
                                jnp.zeros((L,), jnp.float32))
        outv[...] = partial
        pltpu.sync_copy(outv, out_hbm.at[pl.ds(wid * L, L)])

    return sc_loss


_SC_LOSS = _sc_loss_call()


def _rnd16(v):
    i = jax.lax.bitcast_convert_type(v, jnp.int32)
    i = i + 0x7FFF + (jax.lax.shift_right_logical(i, 16) & 1)
    return i & jnp.int32(-65536)


def _pack(a, b):
    return jax.lax.shift_right_logical(_rnd16(a), 16) | _rnd16(b)


def kernel(pos_h, pos_t, pos_r, neg_h, neg_t, neg_r,
           ent_embeddings, rel_embeddings, ent_transfer, rel_transfer):
    ent_pk = _pack(ent_embeddings, ent_transfer)
    rel_pk = _pack(rel_transfer, rel_embeddings)
    h_idx = jnp.concatenate(
        [pos_h.astype(jnp.int32), neg_h.astype(jnp.int32)], axis=1).reshape(-1)
    t_idx = jnp.concatenate(
        [pos_t.astype(jnp.int32), neg_t.astype(jnp.int32)], axis=1).reshape(-1)
    r_idx = jnp.concatenate(
        [pos_r.astype(jnp.int32), neg_r.astype(jnp.int32)], axis=1).reshape(-1)
    partials = _SC_LOSS(h_idx, t_idx, r_idx, ent_pk, rel_pk)
    return jnp.sum(partials[::L])


# speedup vs baseline: 1.1161x; 1.0016x over previous
import functools

import jax
import jax.numpy as jnp
from jax import lax
from jax.experimental import pallas as pl
from jax.experimental.pallas import tpu as pltpu
from jax.experimental.pallas import tpu_sc as plsc

NC = 2
NS = 16
L = 16
NW = NC * NS
B = 16384
NEG = 25
PAIRS = NEG + 1
H = 64
HV = H // L
PW = B // NW * PAIRS
CH_ROWS = 4
CP = CH_ROWS * PAIRS
NCH = PW // CP
MARGIN = 1.0


def _sc_loss_call():
    mesh = plsc.VectorSubcoreMesh(
        core_axis_name="c", subcore_axis_name="s", num_cores=NC)

    @functools.partial(
        pl.kernel,
        mesh=mesh,
        compiler_params=pltpu.CompilerParams(
            use_tc_tiling_on_sc=False, needs_layout_passes=False),
        out_type=jax.ShapeDtypeStruct((NW * L,), jnp.float32),
        scratch_types=[
            pltpu.VMEM((PW,), jnp.int32),
            pltpu.VMEM((PW,), jnp.int32),
            pltpu.VMEM((PW,), jnp.int32),
            pltpu.VMEM((2, CP, H), jnp.int32),
            pltpu.VMEM((2, CP, H), jnp.int32),
            pltpu.VMEM((2, CP, H), jnp.int32),
            pltpu.VMEM((L,), jnp.float32),
            pltpu.SemaphoreType.DMA,
            pltpu.SemaphoreType.DMA,
        ],
    )
    def sc_loss(h_hbm, t_hbm, r_hbm, ent_pk, rel_pk, out_hbm,
                hidx, tidx, ridx, hb, tb, rb, outv, sem0, sem1):
        wid = lax.axis_index("s") * NC + lax.axis_index("c")

        pltpu.sync_copy(h_hbm.at[pl.ds(wid * PW, PW)], hidx)
        pltpu.sync_copy(t_hbm.at[pl.ds(wid * PW, PW)], tidx)
        pltpu.sync_copy(r_hbm.at[pl.ds(wid * PW, PW)], ridx)

        sems = (sem0, sem1)
        bufs = (hb, tb, rb)
        tabs = (ent_pk, ent_pk, rel_pk)
        idxs = (hidx, tidx, ridx)

        def start(g, slot):
            for buf, tab, ix in zip(bufs, tabs, idxs):
                pltpu.async_copy(
                    tab.at[ix.at[pl.ds(g * CP, CP)]], buf.at[slot],
                    sems[slot])

        def drain(slot):
            for buf in bufs:
                pltpu.make_async_copy(
                    ent_pk.at[pl.ds(0, CP)], buf.at[slot],
                    sems[slot]).wait()

        def bsum(v):
            for step in (8, 4, 2, 1):
                idx = lax.iota(jnp.int32, L) ^ step
                v = v + v.at[idx].get(mode="promise_in_bounds")
            return v

        def unpk(v):
            lo = plsc.bitcast(v << 16, jnp.float32)
            hi = plsc.bitcast(v & jnp.int32(-65536), jnp.float32)
            return lo, hi

        def pair_sv(slot, p):
            he, te, q = [], [], None
            for c in range(HV):
                hec, htc = unpk(hb[slot, p, pl.ds(c * L, L)])
                tec, ttc = unpk(tb[slot, p, pl.ds(c * L, L)])
                he.append(hec)
                te.append(tec)
                qc = hec * htc - tec * ttc
                q = qc if q is None else q + qc
            d = jnp.sum(q)
            s = None
            for c in range(HV):
                rtc, rec = unpk(rb[slot, p, pl.ds(c * L, L)])
                sc = jnp.abs(he[c] - te[c] + d * rtc + rec)
                s = sc if s is None else s + sc
            return s

        def compute(g, slot, partial):
            del g

            def row_body(rr, acc):
                base = rr * PAIRS
                ps = pair_sv(slot, base)

                def neg_body(k, nacc):
                    return nacc + pair_sv(slot, base + k)

                nacc = lax.fori_loop(
                    1, PAIRS, neg_body, jnp.zeros((L,), jnp.float32))
                z = bsum(ps) - bsum(nacc) * (1.0 / NEG) + MARGIN
                return acc + jnp.maximum(z, 0.0)

            return lax.fori_loop(0, CH_ROWS, row_body, partial)

        start(0, 0)

        def outer(i, partial):
            g = 2 * i
            start(g + 1, 1)
            drain(0)
            partial = compute(g, 0, partial)

            @pl.when(g + 2 < NCH)
            def _():
                start(g + 2, 0)

            drain(1)
            return compute(g + 1, 1, partial)

        partial = lax.fori_loop(0, NCH // 2, outer,
                                jnp.zeros((L,), jnp.float32))
        outv[...] = partial
        pltpu.sync_copy(outv, out_hbm.at[pl.ds(wid * L, L)])

    return sc_loss


_SC_LOSS = _sc_loss_call()


def _rnd16(v):
    i = jax.lax.bitcast_convert_type(v, jnp.int32)
    i = i + 0x7FFF + (jax.lax.shift_right_logical(i, 16) & 1)
    return i & jnp.int32(-65536)


def _pack(a, b):
    return jax.lax.shift_right_logical(_rnd16(a), 16) | _rnd16(b)


def kernel(pos_h, pos_t, pos_r, neg_h, neg_t, neg_r,
           ent_embeddings, rel_embeddings, ent_transfer, rel_transfer):
    ent_pk = _pack(ent_embeddings, ent_transfer)
    rel_pk = _pack(rel_transfer, rel_embeddings)
    h_idx = jnp.concatenate(
        [pos_h.astype(jnp.int32), neg_h.astype(jnp.int32)], axis=1).reshape(-1)
    t_idx = jnp.concatenate(
        [pos_t.astype(jnp.int32), neg_t.astype(jnp.int32)], axis=1).reshape(-1)
    r_idx = jnp.concatenate(
        [pos_r.astype(jnp.int32), neg_r.astype(jnp.int32)], axis=1).reshape(-1)
    partials = _SC_LOSS(h_idx, t_idx, r_idx, ent_pk, rel_pk)
    return jnp.sum(partials[::L])
